# baseline (speedup 1.0000x reference)
import jax
import jax.numpy as jnp
from jax import lax
from jax.experimental import pallas as pl
from jax.experimental.pallas import tpu as pltpu
from jax.experimental.pallas import tpu_sc as plsc

N = 10000
D = 256
DH = D // 2
L = 16
NS = 16
CHUNK = 128
PAIR = 2 * NS * CHUNK
ROWS_PER_TILE = N // NS
P2 = 64


def _sc_spmm(x2, edata, al, out, acc,
             eb0, eb1, dsx, rw0, rw1, ixb, ab,
             sem_g, sem_ld, sem_sc):
    c = lax.axis_index("c")
    s = lax.axis_index("s")
    nch = edata.shape[0] // NS
    n0 = s * ROWS_PER_TILE
    gc0 = s * nch

    zero = jnp.zeros((L,), jnp.float32)

    @pl.loop(0, CHUNK)
    def _zero_rows(i):
        for j in range(DH // L):
            rw0[i, pl.ds(j * L, L)] = zero

    for i in range(ROWS_PER_TILE // CHUNK):
        pltpu.sync_copy(rw0,
                        acc.at[pl.ds(n0 + i * CHUNK, CHUNK)])
    _rem = ROWS_PER_TILE % CHUNK
    pltpu.sync_copy(rw0.at[pl.ds(0, _rem)],
                    acc.at[pl.ds(n0 + ROWS_PER_TILE - _rem, _rem)])
    plsc.subcore_barrier()

    def adjust(eb):
        for j in range(CHUNK // L):
            sl = pl.ds(j * L, L)
            iv = eb[0, sl]
            eb[0, sl] = iv * 2 + ((iv + c) & 1)

    def scale(eb, rw):
        @pl.loop(0, CHUNK // L)
        def _scale(gg):
            wv = plsc.bitcast(eb[2, pl.ds(gg * L, L)], jnp.float32)
            for k in range(L):
                e = gg * L + k
                w = wv[k]
                for j in range(DH // L):
                    sl = pl.ds(j * L, L)
                    rw[e, sl] = rw[e, sl] * w

    pltpu.sync_copy(edata.at[gc0], eb0)
    adjust(eb0)
    pltpu.async_copy(edata.at[gc0 + 1], eb1, sem_ld)
    pltpu.async_copy(x2.at[eb0.at[0]], rw0, sem_g)

    last = nch - 1

    def half_step(gp, b):
        g = 2 * gp + b
        if b == 0:
            eb_b, eb_n, rw_b, rw_n = eb0, eb1, rw0, rw1
        else:
            eb_b, eb_n, rw_b, rw_n = eb1, eb0, rw1, rw0
        pltpu.make_async_copy(edata.at[gc0], eb_n, sem_ld).wait()
        adjust(eb_n)
        if b == 0:
            @pl.when(gp > 0)
            def _():
                pltpu.make_async_copy(
                    rw_n, acc.at[dsx.at[1 - b]], sem_sc).wait()
        else:
            pltpu.make_async_copy(
                rw_n, acc.at[dsx.at[1 - b]], sem_sc).wait()
        pltpu.async_copy(x2.at[eb_n.at[0]], rw_n, sem_g)
        pltpu.make_async_copy(x2.at[eb_b.at[0]], rw_b, sem_g).wait()
        for j in range(CHUNK // L):
            sl = pl.ds(j * L, L)
            dsx[b, sl] = eb_b[1, sl]
        scale(eb_b, rw_b)
        nxt = jnp.minimum(g + 2, last)
        pltpu.async_copy(edata.at[gc0 + nxt], eb_b, sem_ld)
        pltpu.async_copy(rw_b, acc.at[dsx.at[b]], sem_sc, add=True)

    @pl.loop(0, nch // 2)
    def _pairs(gp):
        half_step(gp, 0)
        half_step(gp, 1)

    pltpu.make_async_copy(x2.at[eb0.at[0]], rw0, sem_g).wait()
    pltpu.make_async_copy(edata.at[gc0], eb1, sem_ld).wait()
    pltpu.make_async_copy(rw1, acc.at[dsx.at[1]], sem_sc).wait()
    plsc.subcore_barrier()

    pltpu.sync_copy(al, ab)
    a_v = ab[...]
    one_minus_a = 1.0 - a_v
    ar1 = jnp.arange(L, dtype=jnp.int32)
    nfull = ROWS_PER_TILE // P2
    rem2 = ROWS_PER_TILE - nfull * P2
    for i in range(nfull + 1):
        r0 = n0 + i * P2
        m = P2 if i < nfull else rem2
        for q in range(P2 // L):
            nid = ar1 + (r0 + q * L)
            ixb[pl.ds(q * L, L)] = nid * 2 + ((nid + c) & 1)
        pltpu.async_copy(x2.at[ixb.at[pl.ds(0, m)]],
                         rw1.at[pl.ds(0, m)], sem_g)
        pltpu.sync_copy(acc.at[pl.ds(r0, m)], rw0.at[pl.ds(0, m)])
        pltpu.make_async_copy(x2.at[ixb.at[pl.ds(0, m)]],
                              rw1.at[pl.ds(0, m)], sem_g).wait()

        @pl.loop(0, m)
        def _combine(r):
            for j in range(DH // L):
                sl = pl.ds(j * L, L)
                rw0[r, sl] = (a_v * rw1[r, sl]
                              + one_minus_a * rw0[r, sl])

        pltpu.sync_copy(rw0.at[pl.ds(0, m)], out.at[pl.ds(r0, m), c])


def kernel(x, edge_index, edge_weight, alpha):
    E = edge_index.shape[1]
    ep = ((E + PAIR - 1) // PAIR) * PAIR
    pad = ep - E
    src = edge_index[0]
    dst = edge_index[1]
    wbits = lax.bitcast_convert_type(edge_weight, jnp.int32)
    if pad:
        src = jnp.concatenate([src, jnp.zeros((pad,), jnp.int32)])
        dst = jnp.concatenate([dst, jnp.zeros((pad,), jnp.int32)])
        wbits = jnp.concatenate([wbits, jnp.zeros((pad,), jnp.int32)])
    nc = ep // CHUNK
    edata = jnp.stack([src.reshape(nc, CHUNK),
                       dst.reshape(nc, CHUNK),
                       wbits.reshape(nc, CHUNK)], axis=1)
    x4 = x.reshape(N, 2, DH)
    odd = (jnp.arange(N, dtype=jnp.int32) & 1).astype(bool)
    x2 = jnp.where(odd[:, None, None], x4[:, ::-1, :], x4).reshape(2 * N, DH)
    al = jnp.broadcast_to(alpha.astype(jnp.float32), (L,))

    mesh = plsc.VectorSubcoreMesh(core_axis_name="c", subcore_axis_name="s")
    out = pl.kernel(
        _sc_spmm,
        out_type=jax.ShapeDtypeStruct((N, 2, DH), jnp.float32),
        mesh=mesh,
        compiler_params=pltpu.CompilerParams(needs_layout_passes=False),
        scratch_types=[
            pltpu.VMEM_SHARED((N, DH), jnp.float32),
            pltpu.VMEM((3, CHUNK), jnp.int32),
            pltpu.VMEM((3, CHUNK), jnp.int32),
            pltpu.VMEM((2, CHUNK), jnp.int32),
            pltpu.VMEM((CHUNK, DH), jnp.float32),
            pltpu.VMEM((CHUNK, DH), jnp.float32),
            pltpu.VMEM((P2,), jnp.int32),
            pltpu.VMEM((L,), jnp.float32),
            pltpu.SemaphoreType.DMA,
            pltpu.SemaphoreType.DMA,
            pltpu.SemaphoreType.DMA,
        ],
    )(x2, edata, al)
    return out.reshape(N, D)


# speedup vs baseline: 1.2422x; 1.2422x over previous
import jax
import jax.numpy as jnp
from jax import lax
from jax.experimental import pallas as pl
from jax.experimental.pallas import tpu as pltpu
from jax.experimental.pallas import tpu_sc as plsc

N = 10000
D = 256
DH = D // 2
L = 16
NS = 16
CHUNK = 128
PAIR = 2 * NS * CHUNK
ROWS_PER_TILE = N // NS
P2 = 64


def _sc_spmm(x2, edata, al, out, acc,
             eb0, eb1, dsx, rw0, rw1, ixb, ab,
             sem_g, sem_ld, sem_sc):
    c = lax.axis_index("c")
    s = lax.axis_index("s")
    nch = edata.shape[0] // NS
    n0 = s * ROWS_PER_TILE
    gc0 = s * nch

    zero = jnp.zeros((L,), jnp.float32)

    @pl.loop(0, CHUNK)
    def _zero_rows(i):
        for j in range(DH // L):
            rw0[i, pl.ds(j * L, L)] = zero

    for i in range(ROWS_PER_TILE // CHUNK):
        pltpu.sync_copy(rw0,
                        acc.at[pl.ds(n0 + i * CHUNK, CHUNK)])
    _rem = ROWS_PER_TILE % CHUNK
    pltpu.sync_copy(rw0.at[pl.ds(0, _rem)],
                    acc.at[pl.ds(n0 + ROWS_PER_TILE - _rem, _rem)])
    plsc.subcore_barrier()

    def adjust(eb):
        for j in range(CHUNK // L):
            sl = pl.ds(j * L, L)
            eb[0, sl] = eb[0, sl] * 2 + c

    def scale(eb, rw):
        @pl.loop(0, CHUNK // L)
        def _scale(gg):
            wv = plsc.bitcast(eb[2, pl.ds(gg * L, L)], jnp.float32)
            for k in range(L):
                e = gg * L + k
                w = wv[k]
                for j in range(DH // L):
                    sl = pl.ds(j * L, L)
                    rw[e, sl] = rw[e, sl] * w

    pltpu.sync_copy(edata.at[gc0], eb0)
    adjust(eb0)
    pltpu.async_copy(edata.at[gc0 + 1], eb1, sem_ld)
    pltpu.async_copy(x2.at[eb0.at[0]], rw0, sem_g)

    last = nch - 1

    def half_step(gp, b):
        g = 2 * gp + b
        if b == 0:
            eb_b, eb_n, rw_b, rw_n = eb0, eb1, rw0, rw1
        else:
            eb_b, eb_n, rw_b, rw_n = eb1, eb0, rw1, rw0
        pltpu.make_async_copy(edata.at[gc0], eb_n, sem_ld).wait()
        adjust(eb_n)
        if b == 0:
            @pl.when(gp > 0)
            def _():
                pltpu.make_async_copy(
                    rw_n, acc.at[dsx.at[1 - b]], sem_sc).wait()
        else:
            pltpu.make_async_copy(
                rw_n, acc.at[dsx.at[1 - b]], sem_sc).wait()
        pltpu.async_copy(x2.at[eb_n.at[0]], rw_n, sem_g)
        pltpu.make_async_copy(x2.at[eb_b.at[0]], rw_b, sem_g).wait()
        for j in range(CHUNK // L):
            sl = pl.ds(j * L, L)
            dsx[b, sl] = eb_b[1, sl]
        scale(eb_b, rw_b)
        nxt = jnp.minimum(g + 2, last)
        pltpu.async_copy(edata.at[gc0 + nxt], eb_b, sem_ld)
        pltpu.async_copy(rw_b, acc.at[dsx.at[b]], sem_sc, add=True)

    @pl.loop(0, nch // 2)
    def _pairs(gp):
        half_step(gp, 0)
        half_step(gp, 1)

    pltpu.make_async_copy(x2.at[eb0.at[0]], rw0, sem_g).wait()
    pltpu.make_async_copy(edata.at[gc0], eb1, sem_ld).wait()
    pltpu.make_async_copy(rw1, acc.at[dsx.at[1]], sem_sc).wait()
    plsc.subcore_barrier()

    pltpu.sync_copy(al, ab)
    a_v = ab[...]
    one_minus_a = 1.0 - a_v
    ar2 = jnp.arange(0, 2 * L, 2, dtype=jnp.int32)
    nfull = ROWS_PER_TILE // P2
    rem2 = ROWS_PER_TILE - nfull * P2
    for i in range(nfull + 1):
        r0 = n0 + i * P2
        m = P2 if i < nfull else rem2
        base = 2 * r0 + c
        for q in range(P2 // L):
            ixb[pl.ds(q * L, L)] = ar2 + (base + 2 * q * L)
        pltpu.async_copy(x2.at[ixb.at[pl.ds(0, m)]],
                         rw1.at[pl.ds(0, m)], sem_g)
        pltpu.sync_copy(acc.at[pl.ds(r0, m)], rw0.at[pl.ds(0, m)])
        pltpu.make_async_copy(x2.at[ixb.at[pl.ds(0, m)]],
                              rw1.at[pl.ds(0, m)], sem_g).wait()

        @pl.loop(0, m)
        def _combine(r):
            for j in range(DH // L):
                sl = pl.ds(j * L, L)
                rw0[r, sl] = (a_v * rw1[r, sl]
                              + one_minus_a * rw0[r, sl])

        pltpu.sync_copy(rw0.at[pl.ds(0, m)], out.at[pl.ds(r0, m), c])


def kernel(x, edge_index, edge_weight, alpha):
    E = edge_index.shape[1]
    ep = ((E + PAIR - 1) // PAIR) * PAIR
    pad = ep - E
    src = edge_index[0]
    dst = edge_index[1]
    wbits = lax.bitcast_convert_type(edge_weight, jnp.int32)
    if pad:
        src = jnp.concatenate([src, jnp.zeros((pad,), jnp.int32)])
        dst = jnp.concatenate([dst, jnp.zeros((pad,), jnp.int32)])
        wbits = jnp.concatenate([wbits, jnp.zeros((pad,), jnp.int32)])
    nc = ep // CHUNK
    edata = jnp.stack([src.reshape(nc, CHUNK),
                       dst.reshape(nc, CHUNK),
                       wbits.reshape(nc, CHUNK)], axis=1)
    x2 = x.reshape(2 * N, DH)
    al = jnp.broadcast_to(alpha.astype(jnp.float32), (L,))

    mesh = plsc.VectorSubcoreMesh(core_axis_name="c", subcore_axis_name="s")
    out = pl.kernel(
        _sc_spmm,
        out_type=jax.ShapeDtypeStruct((N, 2, DH), jnp.float32),
        mesh=mesh,
        compiler_params=pltpu.CompilerParams(needs_layout_passes=False),
        scratch_types=[
            pltpu.VMEM_SHARED((N, DH), jnp.float32),
            pltpu.VMEM((3, CHUNK), jnp.int32),
            pltpu.VMEM((3, CHUNK), jnp.int32),
            pltpu.VMEM((2, CHUNK), jnp.int32),
            pltpu.VMEM((CHUNK, DH), jnp.float32),
            pltpu.VMEM((CHUNK, DH), jnp.float32),
            pltpu.VMEM((P2,), jnp.int32),
            pltpu.VMEM((L,), jnp.float32),
            pltpu.SemaphoreType.DMA,
            pltpu.SemaphoreType.DMA,
            pltpu.SemaphoreType.DMA,
        ],
    )(x2, edata, al)
    return out.reshape(N, D)


# speedup vs baseline: 1.2675x; 1.0204x over previous
import jax
import jax.numpy as jnp
from jax import lax
from jax.experimental import pallas as pl
from jax.experimental.pallas import tpu as pltpu
from jax.experimental.pallas import tpu_sc as plsc

N = 10000
D = 256
DH = D // 2
L = 16
NS = 16
CHUNK = 128
PAIR = 2 * NS * CHUNK
ROWS_PER_TILE = N // NS
P2 = 64


def _sc_spmm(x2, edata, al, out, acc,
             eb0, eb1, dsx, rw0, rw1, ixb, ab,
             sem_g, sem_ld, sem_sc):
    c = lax.axis_index("c")
    s = lax.axis_index("s")
    nch = edata.shape[0] // NS
    n0 = s * ROWS_PER_TILE
    gc0 = s * nch

    zero = jnp.zeros((L,), jnp.float32)

    @pl.loop(0, CHUNK)
    def _zero_rows(i):
        for j in range(DH // L):
            rw0[i, pl.ds(j * L, L)] = zero

    for i in range(ROWS_PER_TILE // CHUNK):
        pltpu.sync_copy(rw0,
                        acc.at[pl.ds(n0 + i * CHUNK, CHUNK)])
    _rem = ROWS_PER_TILE % CHUNK
    pltpu.sync_copy(rw0.at[pl.ds(0, _rem)],
                    acc.at[pl.ds(n0 + ROWS_PER_TILE - _rem, _rem)])
    plsc.subcore_barrier()

    def adjust(eb):
        for j in range(CHUNK // L):
            sl = pl.ds(j * L, L)
            eb[0, sl] = eb[0, sl] * 2 + c

    def scale(eb, rw):
        @pl.loop(0, CHUNK // L)
        def _scale(gg):
            wv = plsc.bitcast(eb[2, pl.ds(gg * L, L)], jnp.float32)
            for k in range(L):
                e = gg * L + k
                w = wv[k]
                for j in range(DH // L):
                    sl = pl.ds(j * L, L)
                    rw[e, sl] = rw[e, sl] * w

    pltpu.sync_copy(edata.at[gc0], eb0)
    adjust(eb0)
    pltpu.async_copy(edata.at[gc0 + 1], eb1, sem_ld)
    pltpu.async_copy(x2.at[eb0.at[0]], rw0, sem_g)

    last = nch - 1

    def half_step(gp, b):
        g = 2 * gp + b
        if b == 0:
            eb_b, eb_n, rw_b, rw_n = eb0, eb1, rw0, rw1
        else:
            eb_b, eb_n, rw_b, rw_n = eb1, eb0, rw1, rw0
        pltpu.make_async_copy(edata.at[gc0], eb_n, sem_ld).wait()
        adjust(eb_n)
        if b == 0:
            @pl.when(gp > 0)
            def _():
                pltpu.make_async_copy(
                    rw_n, acc.at[dsx.at[1 - b]], sem_sc).wait()
        else:
            pltpu.make_async_copy(
                rw_n, acc.at[dsx.at[1 - b]], sem_sc).wait()
        pltpu.async_copy(x2.at[eb_n.at[0]], rw_n, sem_g)
        pltpu.make_async_copy(x2.at[eb_b.at[0]], rw_b, sem_g).wait()
        for j in range(CHUNK // L):
            sl = pl.ds(j * L, L)
            dsx[b, sl] = eb_b[1, sl]
        scale(eb_b, rw_b)
        nxt = jnp.minimum(g + 2, last)
        pltpu.async_copy(edata.at[gc0 + nxt], eb_b, sem_ld)
        pltpu.async_copy(rw_b, acc.at[dsx.at[b]], sem_sc, add=True)

    @pl.loop(0, nch // 2)
    def _pairs(gp):
        half_step(gp, 0)
        half_step(gp, 1)

    pltpu.make_async_copy(x2.at[eb0.at[0]], rw0, sem_g).wait()
    pltpu.make_async_copy(edata.at[gc0], eb1, sem_ld).wait()
    pltpu.make_async_copy(rw1, acc.at[dsx.at[1]], sem_sc).wait()
    plsc.subcore_barrier()

    pltpu.sync_copy(al, ab)
    a_v = ab[...]
    one_minus_a = 1.0 - a_v
    ar2 = jnp.arange(0, 2 * L, 2, dtype=jnp.int32)
    nfull = ROWS_PER_TILE // P2
    rem2 = ROWS_PER_TILE - nfull * P2
    steps = nfull + 1

    def p2_m(i):
        return P2 if i < nfull else rem2

    def p2_issue_gather(i):
        b2 = i % 2
        base = 2 * (n0 + i * P2) + c
        for q in range(P2 // L):
            ixb[b2, pl.ds(q * L, L)] = ar2 + (base + 2 * q * L)
        pltpu.async_copy(x2.at[ixb.at[b2, pl.ds(0, p2_m(i))]],
                         rw1.at[pl.ds(b2 * P2, p2_m(i))], sem_g)

    p2_issue_gather(0)
    for i in range(steps):
        h = (i % 2) * P2
        m = p2_m(i)
        r0 = n0 + i * P2
        if i + 1 < steps:
            p2_issue_gather(i + 1)
        if i >= 2:
            pltpu.make_async_copy(
                rw0.at[pl.ds(h, p2_m(i - 2))],
                out.at[pl.ds(n0 + (i - 2) * P2, p2_m(i - 2)), c],
                sem_ld).wait()
        pltpu.sync_copy(acc.at[pl.ds(r0, m)], rw0.at[pl.ds(h, m)])
        pltpu.make_async_copy(x2.at[ixb.at[i % 2, pl.ds(0, m)]],
                              rw1.at[pl.ds(h, m)], sem_g).wait()

        @pl.loop(0, m)
        def _combine(r):
            for j in range(DH // L):
                sl = pl.ds(j * L, L)
                rw0[h + r, sl] = (a_v * rw1[h + r, sl]
                                  + one_minus_a * rw0[h + r, sl])

        pltpu.async_copy(rw0.at[pl.ds(h, m)],
                         out.at[pl.ds(r0, m), c], sem_ld)
    for i in (steps - 2, steps - 1):
        pltpu.make_async_copy(
            rw0.at[pl.ds((i % 2) * P2, p2_m(i))],
            out.at[pl.ds(n0 + i * P2, p2_m(i)), c],
            sem_ld).wait()


def kernel(x, edge_index, edge_weight, alpha):
    E = edge_index.shape[1]
    ep = ((E + PAIR - 1) // PAIR) * PAIR
    pad = ep - E
    src = edge_index[0]
    dst = edge_index[1]
    wbits = lax.bitcast_convert_type(edge_weight, jnp.int32)
    if pad:
        src = jnp.concatenate([src, jnp.zeros((pad,), jnp.int32)])
        dst = jnp.concatenate([dst, jnp.zeros((pad,), jnp.int32)])
        wbits = jnp.concatenate([wbits, jnp.zeros((pad,), jnp.int32)])
    nc = ep // CHUNK
    edata = jnp.stack([src.reshape(nc, CHUNK),
                       dst.reshape(nc, CHUNK),
                       wbits.reshape(nc, CHUNK)], axis=1)
    x2 = x.reshape(2 * N, DH)
    al = jnp.broadcast_to(alpha.astype(jnp.float32), (L,))

    mesh = plsc.VectorSubcoreMesh(core_axis_name="c", subcore_axis_name="s")
    out = pl.kernel(
        _sc_spmm,
        out_type=jax.ShapeDtypeStruct((N, 2, DH), jnp.float32),
        mesh=mesh,
        compiler_params=pltpu.CompilerParams(needs_layout_passes=False),
        scratch_types=[
            pltpu.VMEM_SHARED((N, DH), jnp.float32),
            pltpu.VMEM((3, CHUNK), jnp.int32),
            pltpu.VMEM((3, CHUNK), jnp.int32),
            pltpu.VMEM((2, CHUNK), jnp.int32),
            pltpu.VMEM((CHUNK, DH), jnp.float32),
            pltpu.VMEM((CHUNK, DH), jnp.float32),
            pltpu.VMEM((2, P2), jnp.int32),
            pltpu.VMEM((L,), jnp.float32),
            pltpu.SemaphoreType.DMA,
            pltpu.SemaphoreType.DMA,
            pltpu.SemaphoreType.DMA,
        ],
    )(x2, edata, al)
    return out.reshape(N, D)
